# Initial kernel scaffold; baseline (speedup 1.0000x reference)
import jax
import jax.numpy as jnp
from jax.experimental import pallas as pl


def kernel(codon_indices, usage_freqs, tai_weights):
    raise NotImplementedError("write your pallas kernel here")


# speedup vs baseline: 4.9317x; 4.9317x over previous
import functools

import jax
import jax.numpy as jnp
from jax import lax
from jax.experimental import pallas as pl
from jax.experimental.pallas import tpu as pltpu
from jax.experimental.pallas import tpu_sc as plsc

L = 16
ROW = 2048
NROWS = 16384
NBINS = 64
CHUNK_ROWS = 16


@functools.lru_cache(maxsize=None)
def _build_sc_kernel(nc, ns):
    nw = nc * ns
    rows_per_w = NROWS // nw
    n_chunks = rows_per_w // CHUNK_ROWS
    chunk_elems = CHUNK_ROWS * ROW

    mesh = plsc.VectorSubcoreMesh(
        core_axis_name="c", subcore_axis_name="s",
        num_cores=nc, num_subcores=ns)

    @functools.partial(
        pl.kernel,
        out_type=(jax.ShapeDtypeStruct((NROWS,), jnp.float32),
                  jax.ShapeDtypeStruct((NROWS,), jnp.float32)),
        mesh=mesh,
        compiler_params=pltpu.CompilerParams(needs_layout_passes=False),
        scratch_types=[
            pltpu.VMEM((chunk_elems,), jnp.int32),
            pltpu.VMEM((L * NBINS,), jnp.float32),
            pltpu.VMEM((NBINS,), jnp.float32),
            pltpu.VMEM((NBINS,), jnp.float32),
            pltpu.VMEM((rows_per_w,), jnp.float32),
            pltpu.VMEM((rows_per_w,), jnp.float32),
            pltpu.VMEM((CHUNK_ROWS * L,), jnp.float32),
            pltpu.VMEM((CHUNK_ROWS * L,), jnp.float32),
        ],
    )
    def k(idx_hbm, tlut_hbm, llut_hbm, tai_hbm, cai_hbm,
          idx_v, hist, tlut, llut, tstage, lstage, tpart, lpart):
        wid = lax.axis_index("s") * nc + lax.axis_index("c")
        base_elem = wid * (rows_per_w * ROW)

        pltpu.sync_copy(tlut_hbm, tlut)
        pltpu.sync_copy(llut_hbm, llut)

        zero16 = jnp.zeros((L,), jnp.float32)
        ones16 = jnp.full((L,), 1.0, jnp.float32)
        lane_base = lax.iota(jnp.int32, L) * NBINS

        for li in range(L):
            for kk in range(NBINS // L):
                hist[pl.ds(li * NBINS + kk * L, L)] = zero16

        def row_body(r, chunk):
            rbase = r * ROW

            def vec_body(j, carry):
                b = rbase + j * (8 * L)
                for u in range(8):
                    iv = idx_v[pl.ds(b + u * L, L)]
                    plsc.addupdate_scatter(hist, [lane_base + iv], ones16)
                return carry

            lax.fori_loop(0, ROW // (8 * L), vec_body, 0, unroll=1)

            bins = [zero16] * (NBINS // L)
            for li in range(L):
                for kk in range(NBINS // L):
                    sl = pl.ds(li * NBINS + kk * L, L)
                    bins[kk] = bins[kk] + hist[sl]
                    hist[sl] = zero16
            tsum = zero16
            lsum = zero16
            for kk in range(NBINS // L):
                tsum = tsum + bins[kk] * tlut[pl.ds(kk * L, L)]
                lsum = lsum + bins[kk] * llut[pl.ds(kk * L, L)]
            tpart[pl.ds(r * L, L)] = tsum
            lpart[pl.ds(r * L, L)] = lsum
            return chunk

        lane_l = lax.iota(jnp.int32, L) * L

        def chunk_body(c, carry):
            pltpu.sync_copy(
                idx_hbm.at[pl.ds(base_elem + c * chunk_elems, chunk_elems)],
                idx_v)
            lax.fori_loop(0, CHUNK_ROWS, row_body, c)
            tvec = zero16
            lvec = zero16
            for li in range(L):
                gi = lane_l + li
                tvec = tvec + plsc.load_gather(tpart, [gi])
                lvec = lvec + plsc.load_gather(lpart, [gi])
            tstage[pl.ds(c * CHUNK_ROWS, CHUNK_ROWS)] = tvec
            lstage[pl.ds(c * CHUNK_ROWS, CHUNK_ROWS)] = lvec
            return carry

        lax.fori_loop(0, n_chunks, chunk_body, 0)

        inv_n = jnp.full((L,), 1.0 / ROW, jnp.float32)

        def post_body(i, carry):
            o = i * L
            tstage[pl.ds(o, L)] = tstage[pl.ds(o, L)] * inv_n
            lstage[pl.ds(o, L)] = jnp.exp(lstage[pl.ds(o, L)] * inv_n)
            return carry

        lax.fori_loop(0, rows_per_w // L, post_body, 0)

        out_base = wid * rows_per_w
        pltpu.sync_copy(tstage, tai_hbm.at[pl.ds(out_base, rows_per_w)])
        pltpu.sync_copy(lstage, cai_hbm.at[pl.ds(out_base, rows_per_w)])

    return k


def kernel(codon_indices, usage_freqs, tai_weights):
    log_lut = jnp.log(usage_freqs / usage_freqs.max() + 1e-8)
    idx_flat = codon_indices.reshape(-1)
    try:
        info = plsc.get_sparse_core_info()
        nc, ns = info.num_cores, info.num_subcores
    except Exception:
        nc, ns = 2, 16
    tai, cai = _build_sc_kernel(nc, ns)(idx_flat, tai_weights, log_lut)
    return tai, cai
